# baseline (speedup 1.0000x reference)
import functools

import jax
import jax.numpy as jnp
from jax import lax
from jax.experimental import pallas as pl
from jax.experimental.pallas import tpu as pltpu
from jax.experimental.pallas import tpu_sc as plsc

B, T, D = 16, 4096, 1024


def _last_row_gather(x_flat, seq_len):
    mesh = plsc.VectorSubcoreMesh(core_axis_name="c", subcore_axis_name="s")

    @functools.partial(
        pl.kernel,
        mesh=mesh,
        out_type=jax.ShapeDtypeStruct((B, D), jnp.float32),
        scratch_types=[
            pltpu.VMEM((B,), jnp.int32),
            pltpu.VMEM((B, D), jnp.float32),
            pltpu.SemaphoreType.DMA,
        ],
    )
    def k(x_hbm, seq_hbm, out_hbm, idx_v, rows_v, sem):
        cid = lax.axis_index("c")
        sid = lax.axis_index("s")

        @pl.when(jnp.logical_and(cid == 0, sid == 0))
        def _():
            pltpu.sync_copy(seq_hbm, idx_v)
            row_ids = lax.iota(jnp.int32, B) * T + idx_v[...] - 1
            idx_v[...] = row_ids
            pltpu.async_copy(x_hbm.at[idx_v], rows_v, sem).wait()
            pltpu.sync_copy(rows_v, out_hbm)

    return k(x_flat, seq_len)


def kernel(x, seq_len):
    x_flat = x.reshape(B * T, D)
    return _last_row_gather(x_flat, seq_len.astype(jnp.int32))


# speedup vs baseline: 1.0398x; 1.0398x over previous
import functools

import jax
import jax.numpy as jnp
from jax import lax
from jax.experimental import pallas as pl
from jax.experimental.pallas import tpu as pltpu
from jax.experimental.pallas import tpu_sc as plsc

B, T, D = 16, 4096, 1024


def _last_row_gather(x, seq_len):
    mesh = plsc.VectorSubcoreMesh(
        core_axis_name="c", subcore_axis_name="s", num_cores=1
    )

    @functools.partial(
        pl.kernel,
        mesh=mesh,
        out_type=jax.ShapeDtypeStruct((B, D), jnp.float32),
        scratch_types=[
            pltpu.VMEM((2 * B,), jnp.int32),
        ],
    )
    def k(x_hbm, seq_hbm, out_hbm, seq_v):
        s = lax.axis_index("s")
        pltpu.sync_copy(seq_hbm, seq_v.at[pl.ds(0, B)])
        t = seq_v[pl.ds(s, B)][0] - 1
        pltpu.sync_copy(x_hbm.at[s].at[pl.ds(t, 1)], out_hbm.at[pl.ds(s, 1)])

    return k(x, seq_len)


def kernel(x, seq_len):
    return _last_row_gather(x, seq_len.astype(jnp.int32))


# speedup vs baseline: 1.1100x; 1.0675x over previous
import functools

import jax
import jax.numpy as jnp
from jax.experimental import pallas as pl
from jax.experimental.pallas import tpu as pltpu
from jax.experimental.pallas import tpu_sc as plsc

B, T, D = 16, 4096, 1024


def _last_row_gather(x, seq_len):
    mesh = plsc.ScalarSubcoreMesh(axis_name="c", num_cores=1)

    @functools.partial(
        pl.kernel,
        mesh=mesh,
        out_type=jax.ShapeDtypeStruct((B, D), jnp.float32),
        scratch_types=[
            pltpu.SMEM((B,), jnp.int32),
            pltpu.SemaphoreType.DMA,
        ],
    )
    def k(x_hbm, seq_hbm, out_hbm, seq_s, sem):
        pltpu.sync_copy(seq_hbm, seq_s)
        copies = []
        for b in range(B):
            t = seq_s[b] - 1
            copies.append(
                pltpu.make_async_copy(
                    x_hbm.at[b].at[pl.ds(t, 1)], out_hbm.at[pl.ds(b, 1)], sem
                )
            )
            copies[-1].start()
        for c in copies:
            c.wait()

    return k(x, seq_len)


def kernel(x, seq_len):
    return _last_row_gather(x, seq_len.astype(jnp.int32))
